# baseline (speedup 1.0000x reference)
import functools

import jax
import jax.numpy as jnp
from jax import lax
from jax.experimental import pallas as pl
from jax.experimental.pallas import tpu as pltpu
from jax.experimental.pallas import tpu_sc as plsc

N = 10000
E = 320000
DIN, DHID, DOUT = 128, 64, 32
D = 128

NC, NS, LANES = 2, 16, 16
NW = NC * NS
CHUNK = 128
NCH = 80
NBUF = 2
E_PAD = NW * NCH * CHUNK
NACC = 10240
RPT = NACC // NS
ZR = 32
BLK = 1024


def _sc_mesh():
    return plsc.VectorSubcoreMesh(
        core_axis_name="c", subcore_axis_name="s",
        num_cores=NC, num_subcores=NS)


def _seg_sum_body(y_hbm, src_hbm, dst_hbm, out_hbm,
                  acc, srcv, dstv, rows, zbuf):
    cid = lax.axis_index("c")
    sid = lax.axis_index("s")
    wid = sid * NC + cid

    zvec = jnp.zeros((LANES,), jnp.float32)

    def zrow(i, carry):
        for k in range(D // LANES):
            zbuf[i, pl.ds(k * LANES, LANES)] = zvec
        return carry

    lax.fori_loop(0, ZR, zrow, 0)

    def zcopy(t, carry):
        pltpu.sync_copy(zbuf, acc.at[pl.ds(sid * RPT + t * ZR, ZR)])
        return carry

    lax.fori_loop(0, RPT // ZR, zcopy, 0)
    plsc.subcore_barrier()

    pltpu.sync_copy(src_hbm.at[wid], srcv)
    pltpu.sync_copy(dst_hbm.at[wid], dstv)

    def chunk_body(j, carry):
        pltpu.sync_copy(y_hbm.at[srcv.at[j]], rows)
        pltpu.sync_copy(rows, acc.at[dstv.at[j]], add=True)
        return carry

    lax.fori_loop(0, NCH, chunk_body, 0)

    plsc.subcore_barrier()
    sl = pl.ds(sid * RPT, RPT)
    pltpu.sync_copy(acc.at[sl], out_hbm.at[cid, sl])


def _seg_sum_sc(y, src3, dst3):
    fn = pl.kernel(
        _seg_sum_body,
        out_type=jax.ShapeDtypeStruct((NC, NACC, D), jnp.float32),
        mesh=_sc_mesh(),
        scratch_types=[
            pltpu.VMEM_SHARED((NACC, D), jnp.float32),
            pltpu.VMEM((NCH, CHUNK), jnp.int32),
            pltpu.VMEM((NCH, CHUNK), jnp.int32),
            pltpu.VMEM((CHUNK, D), jnp.float32),
            pltpu.VMEM((ZR, D), jnp.float32),
        ],
    )
    return fn(y, src3, dst3)


def _deg_body(dst_hbm, out_hbm, accd, dstv, ones, zdeg):
    cid = lax.axis_index("c")
    sid = lax.axis_index("s")
    wid = sid * NC + cid

    zvec = jnp.zeros((LANES,), jnp.float32)
    onev = jnp.ones((LANES,), jnp.float32)

    def orow(i, carry):
        for k in range(D // LANES):
            ones[i, pl.ds(k * LANES, LANES)] = onev
            zdeg[i % ZR, pl.ds(k * LANES, LANES)] = zvec
        return carry

    lax.fori_loop(0, CHUNK, orow, 0)

    def zcopy(t, carry):
        pltpu.sync_copy(zdeg, accd.at[pl.ds(sid * RPT + t * ZR, ZR)])
        return carry

    lax.fori_loop(0, RPT // ZR, zcopy, 0)
    plsc.subcore_barrier()

    pltpu.sync_copy(dst_hbm.at[wid], dstv)

    def chunk_body(j, carry):
        pltpu.sync_copy(ones, accd.at[dstv.at[j]], add=True)
        return carry

    lax.fori_loop(0, NCH, chunk_body, 0)

    plsc.subcore_barrier()
    sl = pl.ds(sid * RPT, RPT)
    pltpu.sync_copy(accd.at[sl], out_hbm.at[cid, sl])


def _deg_sc(dst3):
    fn = pl.kernel(
        _deg_body,
        out_type=jax.ShapeDtypeStruct((NC, NACC, D), jnp.float32),
        mesh=_sc_mesh(),
        scratch_types=[
            pltpu.VMEM_SHARED((NACC, D), jnp.float32),
            pltpu.VMEM((NCH, CHUNK), jnp.int32),
            pltpu.VMEM((CHUNK, D), jnp.float32),
            pltpu.VMEM((ZR, D), jnp.float32),
        ],
    )
    return fn(dst3)


def _tc1_body(s_ref, d_ref, x_ref, wl_ref, wr_ref, b_ref, h_ref):
    s = s_ref[0] + s_ref[1]
    deg = d_ref[0, :, 0:1] + d_ref[1, :, 0:1]
    mean = s / jnp.maximum(deg, 1.0)
    h = jnp.maximum(
        jnp.dot(mean, wl_ref[...], preferred_element_type=jnp.float32)
        + jnp.dot(x_ref[...], wr_ref[...], preferred_element_type=jnp.float32)
        + b_ref[...], 0.0)
    h_ref[...] = jnp.concatenate([h, h], axis=1)


def _tc2_body(s_ref, d_ref, h_ref, wl_ref, wr_ref, b_ref, out_ref):
    s = s_ref[0] + s_ref[1]
    deg = d_ref[0, :, 0:1] + d_ref[1, :, 0:1]
    mean = s / jnp.maximum(deg, 1.0)
    out_ref[...] = (
        jnp.dot(mean, wl_ref[...], preferred_element_type=jnp.float32)
        + jnp.dot(h_ref[...], wr_ref[...], preferred_element_type=jnp.float32)
        + b_ref[...])


def kernel(x, edge_index, W1_l, W1_r, b1, W2_l, W2_r, b2):
    ei = edge_index.astype(jnp.int32)
    pad_dst = N + (jnp.arange(E_PAD - E, dtype=jnp.int32) % (NACC - N))
    src_p = jnp.concatenate([ei[0], jnp.zeros((E_PAD - E,), jnp.int32)])
    dst_p = jnp.concatenate([ei[1], pad_dst])
    src3 = src_p.reshape(NW, NCH, CHUNK)
    dst3 = dst_p.reshape(NW, NCH, CHUNK)

    x_pad = jnp.concatenate(
        [x, jnp.zeros((NACC - N, DIN), jnp.float32)], axis=0)
    b1r = b1.reshape(1, DHID)
    W2l2 = jnp.concatenate([W2_l, W2_l], axis=0) * 0.5
    W2r2 = jnp.concatenate([W2_r, W2_r], axis=0) * 0.5
    b2r = b2.reshape(1, DOUT)

    grid = (NACC // BLK,)

    dpart = _deg_sc(dst3)
    s1 = _seg_sum_sc(x_pad, src3, dst3)

    h2 = pl.pallas_call(
        _tc1_body,
        grid=grid,
        in_specs=[
            pl.BlockSpec((NC, BLK, D), lambda i: (0, i, 0)),
            pl.BlockSpec((NC, BLK, D), lambda i: (0, i, 0)),
            pl.BlockSpec((BLK, DIN), lambda i: (i, 0)),
            pl.BlockSpec((DIN, DHID), lambda i: (0, 0)),
            pl.BlockSpec((DIN, DHID), lambda i: (0, 0)),
            pl.BlockSpec((1, DHID), lambda i: (0, 0)),
        ],
        out_specs=pl.BlockSpec((BLK, D), lambda i: (i, 0)),
        out_shape=jax.ShapeDtypeStruct((NACC, D), jnp.float32),
    )(s1, dpart, x_pad, W1_l, W1_r, b1r)

    s2 = _seg_sum_sc(h2, src3, dst3)

    out = pl.pallas_call(
        _tc2_body,
        grid=grid,
        in_specs=[
            pl.BlockSpec((NC, BLK, D), lambda i: (0, i, 0)),
            pl.BlockSpec((NC, BLK, D), lambda i: (0, i, 0)),
            pl.BlockSpec((BLK, D), lambda i: (i, 0)),
            pl.BlockSpec((D, DOUT), lambda i: (0, 0)),
            pl.BlockSpec((D, DOUT), lambda i: (0, 0)),
            pl.BlockSpec((1, DOUT), lambda i: (0, 0)),
        ],
        out_specs=pl.BlockSpec((BLK, DOUT), lambda i: (i, 0)),
        out_shape=jax.ShapeDtypeStruct((NACC, DOUT), jnp.float32),
    )(s2, dpart, h2, W2l2, W2r2, b2r)

    return out[:N]


# speedup vs baseline: 1.1666x; 1.1666x over previous
import functools

import jax
import jax.numpy as jnp
from jax import lax
from jax.experimental import pallas as pl
from jax.experimental.pallas import tpu as pltpu
from jax.experimental.pallas import tpu_sc as plsc

N = 10000
E = 320000
DIN, DHID, DOUT = 128, 64, 32
D = 128

NC, NS, LANES = 2, 16, 16
NW = NC * NS
CHUNK = 256
NCH = 40
E_PAD = NW * NCH * CHUNK
NACC = 10240
RPT = NACC // NS
ZR = 32
BLK = 1024


def _sc_mesh():
    return plsc.VectorSubcoreMesh(
        core_axis_name="c", subcore_axis_name="s",
        num_cores=NC, num_subcores=NS)


def _seg_sum_body(y_hbm, src_hbm, dst_hbm, out_hbm,
                  acc, srcv, dstv, rows, zbuf):
    cid = lax.axis_index("c")
    sid = lax.axis_index("s")
    wid = sid * NC + cid

    zvec = jnp.zeros((LANES,), jnp.float32)

    def zrow(i, carry):
        for k in range(D // LANES):
            zbuf[i, pl.ds(k * LANES, LANES)] = zvec
        return carry

    lax.fori_loop(0, ZR, zrow, 0)

    def zcopy(t, carry):
        pltpu.sync_copy(zbuf, acc.at[pl.ds(sid * RPT + t * ZR, ZR)])
        return carry

    lax.fori_loop(0, RPT // ZR, zcopy, 0)
    plsc.subcore_barrier()

    def chunk_body(j, carry):
        pltpu.sync_copy(src_hbm.at[wid, j], srcv)
        pltpu.sync_copy(dst_hbm.at[wid, j], dstv)
        pltpu.sync_copy(y_hbm.at[srcv], rows)
        pltpu.sync_copy(rows, acc.at[dstv], add=True)
        return carry

    lax.fori_loop(0, NCH, chunk_body, 0)

    plsc.subcore_barrier()
    sl = pl.ds(sid * RPT, RPT)
    pltpu.sync_copy(acc.at[sl], out_hbm.at[cid, sl])


def _seg_sum_sc(y, src3, dst3):
    fn = pl.kernel(
        _seg_sum_body,
        out_type=jax.ShapeDtypeStruct((NC, NACC, D), jnp.float32),
        mesh=_sc_mesh(),
        scratch_types=[
            pltpu.VMEM_SHARED((NACC, D), jnp.float32),
            pltpu.VMEM((CHUNK,), jnp.int32),
            pltpu.VMEM((CHUNK,), jnp.int32),
            pltpu.VMEM((CHUNK, D), jnp.float32),
            pltpu.VMEM((ZR, D), jnp.float32),
        ],
    )
    return fn(y, src3, dst3)


def _deg_body(dst_hbm, out_hbm, accd, dstv, ones, zdeg):
    cid = lax.axis_index("c")
    sid = lax.axis_index("s")
    wid = sid * NC + cid

    zvec = jnp.zeros((LANES,), jnp.float32)
    onev = jnp.ones((LANES,), jnp.float32)

    def orow(i, carry):
        for k in range(D // LANES):
            ones[i, pl.ds(k * LANES, LANES)] = onev
            zdeg[i % ZR, pl.ds(k * LANES, LANES)] = zvec
        return carry

    lax.fori_loop(0, CHUNK, orow, 0)

    def zcopy(t, carry):
        pltpu.sync_copy(zdeg, accd.at[pl.ds(sid * RPT + t * ZR, ZR)])
        return carry

    lax.fori_loop(0, RPT // ZR, zcopy, 0)
    plsc.subcore_barrier()

    def chunk_body(j, carry):
        pltpu.sync_copy(dst_hbm.at[wid, j], dstv)
        pltpu.sync_copy(ones, accd.at[dstv], add=True)
        return carry

    lax.fori_loop(0, NCH, chunk_body, 0)

    plsc.subcore_barrier()
    sl = pl.ds(sid * RPT, RPT)
    pltpu.sync_copy(accd.at[sl], out_hbm.at[cid, sl])


def _deg_sc(dst3):
    fn = pl.kernel(
        _deg_body,
        out_type=jax.ShapeDtypeStruct((NC, NACC, D), jnp.float32),
        mesh=_sc_mesh(),
        scratch_types=[
            pltpu.VMEM_SHARED((NACC, D), jnp.float32),
            pltpu.VMEM((CHUNK,), jnp.int32),
            pltpu.VMEM((CHUNK, D), jnp.float32),
            pltpu.VMEM((ZR, D), jnp.float32),
        ],
    )
    return fn(dst3)


def _tc1_body(s_ref, d_ref, x_ref, wl_ref, wr_ref, b_ref, h_ref):
    s = s_ref[0] + s_ref[1]
    deg = d_ref[0, :, 0:1] + d_ref[1, :, 0:1]
    mean = s / jnp.maximum(deg, 1.0)
    h = jnp.maximum(
        jnp.dot(mean, wl_ref[...], preferred_element_type=jnp.float32)
        + jnp.dot(x_ref[...], wr_ref[...], preferred_element_type=jnp.float32)
        + b_ref[...], 0.0)
    h_ref[...] = jnp.concatenate([h, h], axis=1)


def _tc2_body(s_ref, d_ref, h_ref, wl_ref, wr_ref, b_ref, out_ref):
    s = s_ref[0] + s_ref[1]
    deg = d_ref[0, :, 0:1] + d_ref[1, :, 0:1]
    mean = s / jnp.maximum(deg, 1.0)
    out_ref[...] = (
        jnp.dot(mean, wl_ref[...], preferred_element_type=jnp.float32)
        + jnp.dot(h_ref[...], wr_ref[...], preferred_element_type=jnp.float32)
        + b_ref[...])


def kernel(x, edge_index, W1_l, W1_r, b1, W2_l, W2_r, b2):
    ei = edge_index.astype(jnp.int32)
    pad_dst = N + (jnp.arange(E_PAD - E, dtype=jnp.int32) % (NACC - N))
    src_p = jnp.concatenate([ei[0], jnp.zeros((E_PAD - E,), jnp.int32)])
    dst_p = jnp.concatenate([ei[1], pad_dst])
    src3 = src_p.reshape(NW, NCH, CHUNK)
    dst3 = dst_p.reshape(NW, NCH, CHUNK)

    x_pad = jnp.concatenate(
        [x, jnp.zeros((NACC - N, DIN), jnp.float32)], axis=0)
    b1r = b1.reshape(1, DHID)
    W2l2 = jnp.concatenate([W2_l, W2_l], axis=0) * 0.5
    W2r2 = jnp.concatenate([W2_r, W2_r], axis=0) * 0.5
    b2r = b2.reshape(1, DOUT)

    grid = (NACC // BLK,)

    dpart = _deg_sc(dst3)
    s1 = _seg_sum_sc(x_pad, src3, dst3)

    h2 = pl.pallas_call(
        _tc1_body,
        grid=grid,
        in_specs=[
            pl.BlockSpec((NC, BLK, D), lambda i: (0, i, 0)),
            pl.BlockSpec((NC, BLK, D), lambda i: (0, i, 0)),
            pl.BlockSpec((BLK, DIN), lambda i: (i, 0)),
            pl.BlockSpec((DIN, DHID), lambda i: (0, 0)),
            pl.BlockSpec((DIN, DHID), lambda i: (0, 0)),
            pl.BlockSpec((1, DHID), lambda i: (0, 0)),
        ],
        out_specs=pl.BlockSpec((BLK, D), lambda i: (i, 0)),
        out_shape=jax.ShapeDtypeStruct((NACC, D), jnp.float32),
    )(s1, dpart, x_pad, W1_l, W1_r, b1r)

    s2 = _seg_sum_sc(h2, src3, dst3)

    out = pl.pallas_call(
        _tc2_body,
        grid=grid,
        in_specs=[
            pl.BlockSpec((NC, BLK, D), lambda i: (0, i, 0)),
            pl.BlockSpec((NC, BLK, D), lambda i: (0, i, 0)),
            pl.BlockSpec((BLK, D), lambda i: (i, 0)),
            pl.BlockSpec((D, DOUT), lambda i: (0, 0)),
            pl.BlockSpec((D, DOUT), lambda i: (0, 0)),
            pl.BlockSpec((1, DOUT), lambda i: (0, 0)),
        ],
        out_specs=pl.BlockSpec((BLK, DOUT), lambda i: (i, 0)),
        out_shape=jax.ShapeDtypeStruct((NACC, DOUT), jnp.float32),
    )(s2, dpart, h2, W2l2, W2r2, b2r)

    return out[:N]


# speedup vs baseline: 2.0094x; 1.7224x over previous
import functools

import jax
import jax.numpy as jnp
from jax import lax
from jax.experimental import pallas as pl
from jax.experimental.pallas import tpu as pltpu
from jax.experimental.pallas import tpu_sc as plsc

N = 10000
E = 320000
DIN, DHID, DOUT = 128, 64, 32
D = 128

NC, NS, LANES = 2, 16, 16
NW = NC * NS
CHUNK = 128
NCH = 80
E_PAD = NW * NCH * CHUNK
NACC = 10240
RPT = NACC // NS
ZR = 32
BLK = 1024


def _sc_mesh():
    return plsc.VectorSubcoreMesh(
        core_axis_name="c", subcore_axis_name="s",
        num_cores=NC, num_subcores=NS)


def _seg_sum_body(y_hbm, src_hbm, dst_hbm, out_hbm,
                  acc, srcv, dstv, rows, zbuf):
    cid = lax.axis_index("c")
    sid = lax.axis_index("s")
    wid = sid * NC + cid

    zvec = jnp.zeros((LANES,), jnp.float32)

    def zrow(i, carry):
        for k in range(D // LANES):
            zbuf[i, pl.ds(k * LANES, LANES)] = zvec
        return carry

    lax.fori_loop(0, ZR, zrow, 0)

    def zcopy(t, carry):
        pltpu.sync_copy(zbuf, acc.at[pl.ds(sid * RPT + t * ZR, ZR)])
        return carry

    lax.fori_loop(0, RPT // ZR, zcopy, 0)
    plsc.subcore_barrier()

    def chunk_body(j, carry):
        pltpu.sync_copy(src_hbm.at[wid, j], srcv)
        pltpu.sync_copy(dst_hbm.at[wid, j], dstv)
        pltpu.sync_copy(y_hbm.at[srcv], rows)
        pltpu.sync_copy(rows, acc.at[dstv], add=True)
        return carry

    lax.fori_loop(0, NCH, chunk_body, 0)

    plsc.subcore_barrier()
    sl = pl.ds(sid * RPT, RPT)
    pltpu.sync_copy(acc.at[sl], out_hbm.at[cid, sl])


def _seg_sum_sc(y, src3, dst3):
    fn = pl.kernel(
        _seg_sum_body,
        out_type=jax.ShapeDtypeStruct((NC, NACC, D), jnp.float32),
        mesh=_sc_mesh(),
        scratch_types=[
            pltpu.VMEM_SHARED((NACC, D), jnp.float32),
            pltpu.VMEM((CHUNK,), jnp.int32),
            pltpu.VMEM((CHUNK,), jnp.int32),
            pltpu.VMEM((CHUNK, D), jnp.float32),
            pltpu.VMEM((ZR, D), jnp.float32),
        ],
    )
    return fn(y, src3, dst3)


def _deg_body(dst_hbm, out_hbm, accd, dstv, ones, zdeg):
    cid = lax.axis_index("c")
    sid = lax.axis_index("s")
    wid = sid * NC + cid

    zvec = jnp.zeros((LANES,), jnp.float32)
    onev = jnp.ones((LANES,), jnp.float32)

    def orow(i, carry):
        for k in range(D // LANES):
            ones[i, pl.ds(k * LANES, LANES)] = onev
            zdeg[i % ZR, pl.ds(k * LANES, LANES)] = zvec
        return carry

    lax.fori_loop(0, CHUNK, orow, 0)

    def zcopy(t, carry):
        pltpu.sync_copy(zdeg, accd.at[pl.ds(sid * RPT + t * ZR, ZR)])
        return carry

    lax.fori_loop(0, RPT // ZR, zcopy, 0)
    plsc.subcore_barrier()

    def chunk_body(j, carry):
        pltpu.sync_copy(dst_hbm.at[wid, j], dstv)
        pltpu.sync_copy(ones, accd.at[dstv], add=True)
        return carry

    lax.fori_loop(0, NCH, chunk_body, 0)

    plsc.subcore_barrier()
    sl = pl.ds(sid * RPT, RPT)
    pltpu.sync_copy(accd.at[sl], out_hbm.at[cid, sl])


def _deg_sc(dst3):
    fn = pl.kernel(
        _deg_body,
        out_type=jax.ShapeDtypeStruct((NC, NACC, D), jnp.float32),
        mesh=_sc_mesh(),
        scratch_types=[
            pltpu.VMEM_SHARED((NACC, D), jnp.float32),
            pltpu.VMEM((CHUNK,), jnp.int32),
            pltpu.VMEM((CHUNK, D), jnp.float32),
            pltpu.VMEM((ZR, D), jnp.float32),
        ],
    )
    return fn(dst3)


def _tc1_body(s_ref, d_ref, x_ref, wl_ref, wr_ref, b_ref, h_ref):
    s = s_ref[0] + s_ref[1]
    deg = d_ref[0, :, 0:1] + d_ref[1, :, 0:1]
    mean = s / jnp.maximum(deg, 1.0)
    h = jnp.maximum(
        jnp.dot(mean, wl_ref[...], preferred_element_type=jnp.float32)
        + jnp.dot(x_ref[...], wr_ref[...], preferred_element_type=jnp.float32)
        + b_ref[...], 0.0)
    h_ref[...] = jnp.concatenate([h, h], axis=1)


def _tc2_body(s_ref, d_ref, h_ref, wl_ref, wr_ref, b_ref, out_ref):
    s = s_ref[0] + s_ref[1]
    deg = d_ref[0, :, 0:1] + d_ref[1, :, 0:1]
    mean = s / jnp.maximum(deg, 1.0)
    out_ref[...] = (
        jnp.dot(mean, wl_ref[...], preferred_element_type=jnp.float32)
        + jnp.dot(h_ref[...], wr_ref[...], preferred_element_type=jnp.float32)
        + b_ref[...])


def kernel(x, edge_index, W1_l, W1_r, b1, W2_l, W2_r, b2):
    ei = edge_index.astype(jnp.int32)
    pad_idx = N + (jnp.arange(E_PAD - E, dtype=jnp.int32) % (NACC - N))
    src_p = jnp.concatenate([ei[0], pad_idx])
    dst_p = jnp.concatenate([ei[1], pad_idx])
    src3 = src_p.reshape(NW, NCH, CHUNK)
    dst3 = dst_p.reshape(NW, NCH, CHUNK)

    x_pad = jnp.concatenate(
        [x, jnp.zeros((NACC - N, DIN), jnp.float32)], axis=0)
    b1r = b1.reshape(1, DHID)
    W2l2 = jnp.concatenate([W2_l, W2_l], axis=0) * 0.5
    W2r2 = jnp.concatenate([W2_r, W2_r], axis=0) * 0.5
    b2r = b2.reshape(1, DOUT)

    grid = (NACC // BLK,)

    dpart = _deg_sc(dst3)
    s1 = _seg_sum_sc(x_pad, src3, dst3)

    h2 = pl.pallas_call(
        _tc1_body,
        grid=grid,
        in_specs=[
            pl.BlockSpec((NC, BLK, D), lambda i: (0, i, 0)),
            pl.BlockSpec((NC, BLK, D), lambda i: (0, i, 0)),
            pl.BlockSpec((BLK, DIN), lambda i: (i, 0)),
            pl.BlockSpec((DIN, DHID), lambda i: (0, 0)),
            pl.BlockSpec((DIN, DHID), lambda i: (0, 0)),
            pl.BlockSpec((1, DHID), lambda i: (0, 0)),
        ],
        out_specs=pl.BlockSpec((BLK, D), lambda i: (i, 0)),
        out_shape=jax.ShapeDtypeStruct((NACC, D), jnp.float32),
    )(s1, dpart, x_pad, W1_l, W1_r, b1r)

    s2 = _seg_sum_sc(h2, src3, dst3)

    out = pl.pallas_call(
        _tc2_body,
        grid=grid,
        in_specs=[
            pl.BlockSpec((NC, BLK, D), lambda i: (0, i, 0)),
            pl.BlockSpec((NC, BLK, D), lambda i: (0, i, 0)),
            pl.BlockSpec((BLK, D), lambda i: (i, 0)),
            pl.BlockSpec((D, DOUT), lambda i: (0, 0)),
            pl.BlockSpec((D, DOUT), lambda i: (0, 0)),
            pl.BlockSpec((1, DOUT), lambda i: (0, 0)),
        ],
        out_specs=pl.BlockSpec((BLK, DOUT), lambda i: (i, 0)),
        out_shape=jax.ShapeDtypeStruct((NACC, DOUT), jnp.float32),
    )(s2, dpart, h2, W2l2, W2r2, b2r)

    return out[:N]
